# baseline (speedup 1.0000x reference)
import functools

import jax
import jax.numpy as jnp
from jax import lax
from jax.experimental import pallas as pl
from jax.experimental.pallas import tpu as pltpu
from jax.experimental.pallas import tpu_sc as plsc

LANES = 16


def kernel(position_ids, cos_cache, sin_cache):
    bsz, seq = position_ids.shape
    total = bsz * seq
    n_pos, head_half = cos_cache.shape
    info = plsc.get_sparse_core_info()
    ns = info.num_subcores
    h_per_w = head_half // ns

    cos_t = cos_cache.T
    sin_t = sin_cache.T

    mesh = plsc.VectorSubcoreMesh(core_axis_name="c", subcore_axis_name="s")

    @functools.partial(
        pl.kernel,
        mesh=mesh,
        out_type=(
            jax.ShapeDtypeStruct((bsz, head_half, seq), jnp.float32),
            jax.ShapeDtypeStruct((bsz, head_half, seq), jnp.float32),
        ),
        scratch_types=[
            pltpu.VMEM_SHARED((bsz, seq), jnp.int32),
            pltpu.VMEM((bsz, seq), jnp.int32),
            pltpu.VMEM((n_pos,), jnp.float32),
            pltpu.VMEM((n_pos,), jnp.float32),
            pltpu.VMEM((seq,), jnp.float32),
            pltpu.VMEM((seq,), jnp.float32),
            pltpu.SemaphoreType.DMA,
            pltpu.SemaphoreType.DMA,
            pltpu.SemaphoreType.DMA,
            pltpu.SemaphoreType.DMA,
        ],
        compiler_params=pltpu.CompilerParams(needs_layout_passes=False),
    )
    def rope_gather(idx_hbm, cos_hbm, sin_hbm, cos_out, sin_out,
                    idx_sh, idx_v, row0, row1, ob0, ob1,
                    isem, rsem0, rsem1, osem):
        core = lax.axis_index("c")
        tile = lax.axis_index("s")
        h0 = tile * h_per_w
        rows = (row0, row1)
        rsems = (rsem0, rsem1)
        obufs = (ob0, ob1)

        def process(tab, outp):
            row_cps = [None] * h_per_w
            out_cps = [None] * (h_per_w * bsz)

            def start_row(j):
                cp = pltpu.make_async_copy(
                    tab.at[h0 + j], rows[j % 2], rsems[j % 2])
                cp.start()
                row_cps[j] = cp

            start_row(0)
            @pl.when(tile == 0)
            def _():
                pltpu.sync_copy(idx_hbm, idx_sh)

            plsc.subcore_barrier()
            idx_cp = pltpu.make_async_copy(idx_sh, idx_v, isem)
            idx_cp.start()
            idx_cp.wait()
            for j in range(h_per_w):
                if j + 1 < h_per_w:
                    start_row(j + 1)
                row_cps[j].wait()
                row = rows[j % 2]
                for b in range(bsz):
                    t = j * bsz + b
                    slot = t % 2
                    if t >= 2:
                        out_cps[t - 2].wait()
                    obs = obufs[slot]

                    def body(s):
                        off = s * LANES
                        iv = idx_v[b, pl.ds(off, LANES)]
                        obs[pl.ds(off, LANES)] = plsc.load_gather(row, [iv])

                    plsc.parallel_loop(0, seq // LANES, 1, unroll=4)(body)
                    cp = pltpu.make_async_copy(
                        obs, outp.at[b, h0 + j], osem)
                    cp.start()
                    out_cps[t] = cp
            out_cps[-2].wait()
            out_cps[-1].wait()

        @pl.when(core == 0)
        def _():
            process(cos_hbm, cos_out)

        @pl.when(core == 1)
        def _():
            process(sin_hbm, sin_out)

    cos_r, sin_r = rope_gather(position_ids, cos_t, sin_t)
    return (
        jnp.transpose(cos_r, (0, 2, 1)),
        jnp.transpose(sin_r, (0, 2, 1)),
    )


# speedup vs baseline: 1.0238x; 1.0238x over previous
import functools

import jax
import jax.numpy as jnp
from jax import lax
from jax.experimental import pallas as pl
from jax.experimental.pallas import tpu as pltpu
from jax.experimental.pallas import tpu_sc as plsc

LANES = 16


def kernel(position_ids, cos_cache, sin_cache):
    bsz, seq = position_ids.shape
    total = bsz * seq
    n_pos, head_half = cos_cache.shape
    info = plsc.get_sparse_core_info()
    ns = info.num_subcores
    h_per_w = head_half // ns

    cos_t = cos_cache.T
    sin_t = sin_cache.T

    mesh = plsc.VectorSubcoreMesh(core_axis_name="c", subcore_axis_name="s")

    @functools.partial(
        pl.kernel,
        mesh=mesh,
        out_type=(
            jax.ShapeDtypeStruct((bsz, head_half, seq), jnp.float32),
            jax.ShapeDtypeStruct((bsz, head_half, seq), jnp.float32),
        ),
        scratch_types=[
            pltpu.VMEM_SHARED((bsz, seq), jnp.int32),
            pltpu.VMEM((bsz, seq), jnp.int32),
            pltpu.VMEM((n_pos,), jnp.float32),
            pltpu.VMEM((n_pos,), jnp.float32),
            pltpu.VMEM((seq,), jnp.float32),
            pltpu.VMEM((seq,), jnp.float32),
            pltpu.SemaphoreType.DMA,
            pltpu.SemaphoreType.DMA,
            pltpu.SemaphoreType.DMA,
            pltpu.SemaphoreType.DMA,
        ],
        compiler_params=pltpu.CompilerParams(
            needs_layout_passes=False, skip_device_barrier=True),
    )
    def rope_gather(idx_hbm, cos_hbm, sin_hbm, cos_out, sin_out,
                    idx_sh, idx_v, row0, row1, ob0, ob1,
                    isem, rsem0, rsem1, osem):
        core = lax.axis_index("c")
        tile = lax.axis_index("s")
        h0 = tile * h_per_w
        rows = (row0, row1)
        rsems = (rsem0, rsem1)
        obufs = (ob0, ob1)

        def process(tab, outp):
            row_cps = [None] * h_per_w
            out_cps = [None] * (h_per_w * bsz)

            def start_row(j):
                cp = pltpu.make_async_copy(
                    tab.at[h0 + j], rows[j % 2], rsems[j % 2])
                cp.start()
                row_cps[j] = cp

            start_row(0)
            @pl.when(tile == 0)
            def _():
                pltpu.sync_copy(idx_hbm, idx_sh)

            plsc.subcore_barrier()
            idx_cp = pltpu.make_async_copy(idx_sh, idx_v, isem)
            idx_cp.start()
            idx_cp.wait()
            for j in range(h_per_w):
                if j + 1 < h_per_w:
                    start_row(j + 1)
                row_cps[j].wait()
                row = rows[j % 2]
                for b in range(bsz):
                    t = j * bsz + b
                    slot = t % 2
                    if t >= 2:
                        out_cps[t - 2].wait()
                    obs = obufs[slot]

                    def body(s):
                        off = s * LANES
                        iv = idx_v[b, pl.ds(off, LANES)]
                        obs[pl.ds(off, LANES)] = plsc.load_gather(row, [iv])

                    plsc.parallel_loop(0, seq // LANES, 1, unroll=8)(body)
                    cp = pltpu.make_async_copy(
                        obs, outp.at[b, h0 + j], osem)
                    cp.start()
                    out_cps[t] = cp
            out_cps[-2].wait()
            out_cps[-1].wait()

        @pl.when(core == 0)
        def _():
            process(cos_hbm, cos_out)

        @pl.when(core == 1)
        def _():
            process(sin_hbm, sin_out)

    cos_r, sin_r = rope_gather(position_ids, cos_t, sin_t)
    return (
        jnp.transpose(cos_r, (0, 2, 1)),
        jnp.transpose(sin_r, (0, 2, 1)),
    )


# speedup vs baseline: 1.0252x; 1.0013x over previous
import functools

import jax
import jax.numpy as jnp
from jax import lax
from jax.experimental import pallas as pl
from jax.experimental.pallas import tpu as pltpu
from jax.experimental.pallas import tpu_sc as plsc

LANES = 16


def kernel(position_ids, cos_cache, sin_cache):
    bsz, seq = position_ids.shape
    total = bsz * seq
    n_pos, head_half = cos_cache.shape
    info = plsc.get_sparse_core_info()
    ns = info.num_subcores
    h_per_w = head_half // ns

    cos_t = cos_cache.T
    sin_t = sin_cache.T

    mesh = plsc.VectorSubcoreMesh(core_axis_name="c", subcore_axis_name="s")

    @functools.partial(
        pl.kernel,
        mesh=mesh,
        out_type=(
            jax.ShapeDtypeStruct((bsz, head_half, seq), jnp.float32),
            jax.ShapeDtypeStruct((bsz, head_half, seq), jnp.float32),
        ),
        scratch_types=[
            pltpu.VMEM_SHARED((bsz, seq), jnp.int32),
            pltpu.VMEM((bsz, seq), jnp.int32),
            pltpu.VMEM((n_pos,), jnp.float32),
            pltpu.VMEM((n_pos,), jnp.float32),
            pltpu.VMEM((seq,), jnp.float32),
            pltpu.VMEM((seq,), jnp.float32),
            pltpu.SemaphoreType.DMA,
            pltpu.SemaphoreType.DMA,
            pltpu.SemaphoreType.DMA,
            pltpu.SemaphoreType.DMA,
        ],
        compiler_params=pltpu.CompilerParams(needs_layout_passes=False),
    )
    def rope_gather(idx_hbm, cos_hbm, sin_hbm, cos_out, sin_out,
                    idx_sh, idx_v, row0, row1, ob0, ob1,
                    isem, rsem0, rsem1, osem):
        core = lax.axis_index("c")
        tile = lax.axis_index("s")
        h0 = tile * h_per_w
        rows = (row0, row1)
        rsems = (rsem0, rsem1)
        obufs = (ob0, ob1)

        def process(tab, outp):
            row_cps = [None] * h_per_w
            out_cps = [None] * (h_per_w * bsz)

            def start_row(j):
                cp = pltpu.make_async_copy(
                    tab.at[h0 + j], rows[j % 2], rsems[j % 2])
                cp.start()
                row_cps[j] = cp

            start_row(0)
            @pl.when(tile == 0)
            def _():
                pltpu.sync_copy(idx_hbm, idx_sh)

            plsc.subcore_barrier()
            idx_cp = pltpu.make_async_copy(idx_sh, idx_v, isem)
            idx_cp.start()
            idx_cp.wait()
            for j in range(h_per_w):
                if j + 1 < h_per_w:
                    start_row(j + 1)
                row_cps[j].wait()
                row = rows[j % 2]
                for b in range(bsz):
                    t = j * bsz + b
                    slot = t % 2
                    if t >= 2:
                        out_cps[t - 2].wait()
                    obs = obufs[slot]

                    def body(s):
                        off = s * LANES
                        iv = idx_v[b, pl.ds(off, LANES)]
                        obs[pl.ds(off, LANES)] = plsc.load_gather(row, [iv])

                    plsc.parallel_loop(0, seq // LANES, 1, unroll=8)(body)
                    cp = pltpu.make_async_copy(
                        obs, outp.at[b, h0 + j], osem)
                    cp.start()
                    out_cps[t] = cp
            out_cps[-2].wait()
            out_cps[-1].wait()

        @pl.when(core == 0)
        def _():
            process(cos_hbm, cos_out)

        @pl.when(core == 1)
        def _():
            process(sin_hbm, sin_out)

    cos_r, sin_r = rope_gather(position_ids, cos_t, sin_t)
    return (
        jnp.transpose(cos_r, (0, 2, 1)),
        jnp.transpose(sin_r, (0, 2, 1)),
    )
